# baseline (speedup 1.0000x reference)
import functools

import jax
import jax.numpy as jnp
from jax import lax
from jax.experimental import pallas as pl
from jax.experimental.pallas import tpu as pltpu
from jax.experimental.pallas import tpu_sc as plsc

T = 16384
D = 1024
S = 512

A1N, A2N, A3N, A4N = 2048, 256, 32, 4
OFF_A1 = 8
OFF_Q1 = OFF_A1 + A1N
OFF_P1 = OFF_Q1 + A1N
OFF_A2 = OFF_P1 + A1N
OFF_Q2 = OFF_A2 + A2N
OFF_P2 = OFF_Q2 + A2N
OFF_A3 = OFF_P2 + A2N
OFF_Q3 = OFF_A3 + A3N
OFF_P3 = OFF_Q3 + A3N
OFF_A4 = OFF_P3 + A3N
G_ROWS = OFF_A4 + 8

NC, NS, L = 2, 16, 16
NW = NC * NS
SEG_PER_W = S // NW
N0 = 16
NG = 32
NROWS = N0 + NG


def _ps8(x):
    n = x.shape[0]
    rid = lax.broadcasted_iota(jnp.int32, x.shape, 0) % 8
    p = x
    q = x
    for d in (1, 2, 4):
        pad = jnp.full((d, x.shape[1]), -jnp.inf, jnp.float32)
        down = jnp.concatenate([pad, p[:n - d]], axis=0)
        p = jnp.where(rid >= d, jnp.maximum(p, down), p)
        up = jnp.concatenate([q[d:], pad], axis=0)
        q = jnp.where(rid < 8 - d, jnp.maximum(q, up), q)
    return p, q


def _build_body(f_ref, g_ref):
    x = f_ref[...]
    a1 = jnp.max(x.reshape(A1N, 8, 128), axis=1)
    a2 = jnp.max(a1.reshape(A2N, 8, 128), axis=1)
    a3 = jnp.max(a2.reshape(A3N, 8, 128), axis=1)
    a4 = jnp.max(a3.reshape(A4N, 8, 128), axis=1)
    p1, q1 = _ps8(a1)
    p2, q2 = _ps8(a2)
    p3, q3 = _ps8(a3)
    neg8 = jnp.full((8, 128), -jnp.inf, jnp.float32)
    g_ref[0:OFF_A1, :] = neg8
    g_ref[OFF_A1:OFF_Q1, :] = a1
    g_ref[OFF_Q1:OFF_P1, :] = q1
    g_ref[OFF_P1:OFF_A2, :] = p1
    g_ref[OFF_A2:OFF_Q2, :] = a2
    g_ref[OFF_Q2:OFF_P2, :] = q2
    g_ref[OFF_P2:OFF_A3, :] = p2
    g_ref[OFF_A3:OFF_Q3, :] = a3
    g_ref[OFF_Q3:OFF_P3, :] = q3
    g_ref[OFF_P3:OFF_A4, :] = p3
    g_ref[OFF_A4:G_ROWS, :] = jnp.concatenate(
        [a4, jnp.full((4, 128), -jnp.inf, jnp.float32)], axis=0)


def _build_table(frame_features):
    return pl.pallas_call(
        _build_body,
        grid=(D // 128,),
        in_specs=[pl.BlockSpec((T, 128), lambda i: (0, i))],
        out_specs=pl.BlockSpec((G_ROWS, 128), lambda i: (0, i)),
        out_shape=jax.ShapeDtypeStruct((G_ROWS, D), jnp.float32),
    )(frame_features)


_LEVELS = ((OFF_A1, OFF_Q1, OFF_P1),
           (OFF_A2, OFF_Q2, OFF_P2),
           (OFF_A3, OFF_Q3, OFF_P3))


def _segmax_body(f_hbm, g_hbm, s_hbm, e_hbm, out_hbm,
                 sv, ev, idx0, idxg, rows, outs,
                 semA0, semB0, semA1, semB1):
    wid = lax.axis_index("s") * NC + lax.axis_index("c")
    base = wid * SEG_PER_W

    pltpu.sync_copy(s_hbm.at[pl.ds(base, SEG_PER_W)], sv)
    pltpu.sync_copy(e_hbm.at[pl.ds(base, SEG_PER_W)], ev)
    s0 = sv[...]
    e0 = ev[...]
    lane = lax.iota(jnp.int32, L)
    zero = jnp.zeros((L,), jnp.int32)

    for t in range(8):
        plsc.store_scatter(idx0, [lane * N0 + t], jnp.minimum(s0 + t, e0))
        plsc.store_scatter(idx0, [lane * N0 + (8 + t)], jnp.maximum(e0 - t, s0))

    sk = (s0 + 7) >> 3
    ek = ((e0 + 1) >> 3) - 1
    pos = 0
    for offA, offQ, offP in _LEVELS:
        empty = sk > ek
        be_s = ((sk >> 3) << 3) + 7
        bs_e = (ek >> 3) << 3
        l0 = jnp.where(empty, 0, jnp.where(be_s <= ek, offQ + sk, offA + sk))
        l1 = jnp.where(empty, 0, jnp.where(bs_e >= sk, offP + ek, offA + ek))
        plsc.store_scatter(idxg, [lane * NG + pos], l0)
        plsc.store_scatter(idxg, [lane * NG + pos + 1], l1)
        for t in range(6):
            it = jnp.where(empty, 0, offA + jnp.minimum(sk + t, ek))
            plsc.store_scatter(idxg, [lane * NG + pos + 2 + t], it)
        sk = (sk >> 3) + 1
        ek = (ek >> 3) - 1
        pos += 8

    empty = sk > ek
    for t in range(4):
        it = jnp.where(empty, 0, OFF_A4 + jnp.minimum(sk + t, ek))
        plsc.store_scatter(idxg, [lane * NG + pos + t], it)
    for t in range(4):
        plsc.store_scatter(idxg, [lane * NG + pos + 4 + t], zero)

    semsA = (semA0, semA1)
    semsB = (semB0, semB1)

    def copies(j, p):
        cf = pltpu.make_async_copy(
            f_hbm.at[idx0.at[pl.ds(j * N0, N0)]],
            rows.at[p, pl.ds(0, N0)], semsA[p])
        cg = pltpu.make_async_copy(
            g_hbm.at[idxg.at[pl.ds(j * NG, NG)]],
            rows.at[p, pl.ds(N0, NG)], semsB[p])
        return cf, cg

    def start(j, p):
        cf, cg = copies(j, p)
        cf.start()
        cg.start()

    def wait(j, p):
        cf, cg = copies(j, p)
        cf.wait()
        cg.wait()

    def reduce_seg(j, p):
        def v_body(v, c):
            vals = [rows[p, r, pl.ds(v * L, L)] for r in range(NROWS)]
            while len(vals) > 1:
                nxt = [jnp.maximum(vals[i], vals[i + 1])
                       for i in range(0, len(vals) - 1, 2)]
                if len(vals) % 2:
                    nxt.append(vals[-1])
                vals = nxt
            outs[j, pl.ds(v * L, L)] = vals[0]
            return c
        lax.fori_loop(0, D // L, v_body, 0)

    start(0, 0)

    def body(i, c):
        j0 = 2 * i
        start(j0 + 1, 1)
        wait(j0, 0)
        reduce_seg(j0, 0)

        @pl.when(i < SEG_PER_W // 2 - 1)
        def _():
            start(j0 + 2, 0)

        wait(j0 + 1, 1)
        reduce_seg(j0 + 1, 1)
        return c

    lax.fori_loop(0, SEG_PER_W // 2, body, 0)
    pltpu.sync_copy(outs, out_hbm.at[pl.ds(base, SEG_PER_W)])


def _segmax(frame_features, g, s_arr, e_arr):
    mesh = plsc.VectorSubcoreMesh(core_axis_name="c", subcore_axis_name="s")
    run = functools.partial(
        pl.kernel,
        out_type=jax.ShapeDtypeStruct((S, D), jnp.float32),
        mesh=mesh,
        compiler_params=pltpu.CompilerParams(needs_layout_passes=False),
        scratch_types=[
            pltpu.VMEM((SEG_PER_W,), jnp.int32),
            pltpu.VMEM((SEG_PER_W,), jnp.int32),
            pltpu.VMEM((SEG_PER_W * N0,), jnp.int32),
            pltpu.VMEM((SEG_PER_W * NG,), jnp.int32),
            pltpu.VMEM((2, NROWS, D), jnp.float32),
            pltpu.VMEM((SEG_PER_W, D), jnp.float32),
            pltpu.SemaphoreType.DMA,
            pltpu.SemaphoreType.DMA,
            pltpu.SemaphoreType.DMA,
            pltpu.SemaphoreType.DMA,
        ],
    )(_segmax_body)
    return run(frame_features, g, s_arr, e_arr)


def _ln(x, gamma, beta):
    mu = jnp.mean(x, axis=-1, keepdims=True)
    var = jnp.mean((x - mu) ** 2, axis=-1, keepdims=True)
    return (x - mu) / jnp.sqrt(var + 1e-6) * gamma + beta


def _mlp_body(x_ref, w1_ref, b1_ref, w2_ref, b2_ref,
              gy_ref, by_ref, gl_ref, bl_ref, out_ref):
    x = x_ref[...]
    y = _ln(x, gy_ref[...], by_ref[...])
    h = jnp.dot(y, w1_ref[...], preferred_element_type=jnp.float32)
    h = jax.nn.relu(h + b1_ref[...])
    h = _ln(h, gl_ref[...], bl_ref[...])
    logits = jnp.sum(h * w2_ref[...], axis=-1) + b2_ref[0, 0]
    out_ref[...] = jax.nn.sigmoid(logits)[None, :]


def _mlp(segs, W1, b1, W2, b2, gamma_y, beta_y, gamma_l, beta_l):
    full = lambda i: (0, 0)
    return pl.pallas_call(
        _mlp_body,
        grid=(S // 128,),
        in_specs=[
            pl.BlockSpec((128, D), lambda i: (i, 0)),
            pl.BlockSpec((D, D), full),
            pl.BlockSpec((1, D), full),
            pl.BlockSpec((1, D), full),
            pl.BlockSpec((1, 1), full),
            pl.BlockSpec((1, D), full),
            pl.BlockSpec((1, D), full),
            pl.BlockSpec((1, D), full),
            pl.BlockSpec((1, D), full),
        ],
        out_specs=pl.BlockSpec((1, 128), lambda i: (0, i)),
        out_shape=jax.ShapeDtypeStruct((1, S), jnp.float32),
    )(segs, W1, b1.reshape(1, D), W2.reshape(1, D), b2.reshape(1, 1),
      gamma_y.reshape(1, D), beta_y.reshape(1, D),
      gamma_l.reshape(1, D), beta_l.reshape(1, D))


def kernel(frame_features, W1, b1, W2, b2,
           gamma_y, beta_y, gamma_l, beta_l, change_point):
    cp = change_point.astype(jnp.int32)
    s_arr = cp[:, 0]
    e_arr = cp[:, 1]
    g = _build_table(frame_features)
    segs = _segmax(frame_features, g, s_arr, e_arr)
    return _mlp(segs, W1, b1, W2, b2, gamma_y, beta_y, gamma_l, beta_l)


# speedup vs baseline: 1.6804x; 1.6804x over previous
import functools

import jax
import jax.numpy as jnp
from jax import lax
from jax.experimental import pallas as pl
from jax.experimental.pallas import tpu as pltpu
from jax.experimental.pallas import tpu_sc as plsc

T = 16384
D = 1024
S = 512

A1N, A2N, A3N, A4N = 2048, 256, 32, 4
OFF_A1 = 8
OFF_Q1 = OFF_A1 + A1N
OFF_P1 = OFF_Q1 + A1N
OFF_A2 = OFF_P1 + A1N
OFF_Q2 = OFF_A2 + A2N
OFF_P2 = OFF_Q2 + A2N
OFF_A3 = OFF_P2 + A2N
OFF_Q3 = OFF_A3 + A3N
OFF_P3 = OFF_Q3 + A3N
OFF_A4 = OFF_P3 + A3N
SENT = OFF_A4 + 8
G_ROWS = SENT + 32

NC, NS, L = 2, 16, 16
NW = NC * NS
SEG_PER_W = S // NW
N0 = 16
NG = 32
NROWS = N0 + NG


def _ps8(x):
    n = x.shape[0]
    rid = lax.broadcasted_iota(jnp.int32, x.shape, 0) % 8
    p = x
    q = x
    for d in (1, 2, 4):
        pad = jnp.full((d, x.shape[1]), -jnp.inf, jnp.float32)
        down = jnp.concatenate([pad, p[:n - d]], axis=0)
        p = jnp.where(rid >= d, jnp.maximum(p, down), p)
        up = jnp.concatenate([q[d:], pad], axis=0)
        q = jnp.where(rid < 8 - d, jnp.maximum(q, up), q)
    return p, q


def _build_body(f_ref, g_ref):
    x = f_ref[...]
    a1 = jnp.max(x.reshape(A1N, 8, 128), axis=1)
    a2 = jnp.max(a1.reshape(A2N, 8, 128), axis=1)
    a3 = jnp.max(a2.reshape(A3N, 8, 128), axis=1)
    a4 = jnp.max(a3.reshape(A4N, 8, 128), axis=1)
    p1, q1 = _ps8(a1)
    p2, q2 = _ps8(a2)
    p3, q3 = _ps8(a3)
    neg8 = jnp.full((8, 128), -jnp.inf, jnp.float32)
    g_ref[0:OFF_A1, :] = neg8
    g_ref[OFF_A1:OFF_Q1, :] = a1
    g_ref[OFF_Q1:OFF_P1, :] = q1
    g_ref[OFF_P1:OFF_A2, :] = p1
    g_ref[OFF_A2:OFF_Q2, :] = a2
    g_ref[OFF_Q2:OFF_P2, :] = q2
    g_ref[OFF_P2:OFF_A3, :] = p2
    g_ref[OFF_A3:OFF_Q3, :] = a3
    g_ref[OFF_Q3:OFF_P3, :] = q3
    g_ref[OFF_P3:OFF_A4, :] = p3
    g_ref[OFF_A4:G_ROWS, :] = jnp.concatenate(
        [a4, jnp.full((G_ROWS - OFF_A4 - A4N, 128), -jnp.inf, jnp.float32)],
        axis=0)


def _build_table(frame_features):
    return pl.pallas_call(
        _build_body,
        grid=(D // 128,),
        in_specs=[pl.BlockSpec((T, 128), lambda i: (0, i))],
        out_specs=pl.BlockSpec((G_ROWS, 128), lambda i: (0, i)),
        out_shape=jax.ShapeDtypeStruct((G_ROWS, D), jnp.float32),
    )(frame_features)


_LEVELS = ((OFF_A1, OFF_Q1, OFF_P1),
           (OFF_A2, OFF_Q2, OFF_P2),
           (OFF_A3, OFF_Q3, OFF_P3))


def _segmax_body(f_hbm, g_hbm, s_hbm, e_hbm, out_hbm,
                 sv, ev, idx0, idxg, rows, outs,
                 semA0, semB0, semA1, semB1):
    wid = lax.axis_index("s") * NC + lax.axis_index("c")
    base = wid * SEG_PER_W

    pltpu.sync_copy(s_hbm.at[pl.ds(base, SEG_PER_W)], sv)
    pltpu.sync_copy(e_hbm.at[pl.ds(base, SEG_PER_W)], ev)
    s0 = sv[...]
    e0 = ev[...]
    lane = lax.iota(jnp.int32, L)
    zero = jnp.zeros((L,), jnp.int32)

    for t in range(8):
        plsc.store_scatter(idx0, [lane * N0 + t], jnp.minimum(s0 + t, e0))
        plsc.store_scatter(idx0, [lane * N0 + (8 + t)], jnp.maximum(e0 - t, s0))

    sk = (s0 + 7) >> 3
    ek = ((e0 + 1) >> 3) - 1
    pos = 0
    for offA, offQ, offP in _LEVELS:
        nonempty = sk <= ek
        be_s = ((sk >> 3) << 3) + 7
        bs_e = (ek >> 3) << 3
        qv = jnp.logical_and(nonempty, be_s <= ek)
        pv = jnp.logical_and(nonempty, bs_e >= sk)
        interior = jnp.logical_and(
            nonempty,
            jnp.logical_and(jnp.logical_not(qv), jnp.logical_not(pv)))
        l0 = jnp.where(qv, offQ + sk, SENT + pos)
        l1 = jnp.where(pv, offP + ek, SENT + pos + 1)
        plsc.store_scatter(idxg, [lane * NG + pos], l0)
        plsc.store_scatter(idxg, [lane * NG + pos + 1], l1)
        for t in range(6):
            ok = jnp.logical_and(interior, sk + t <= ek)
            it = jnp.where(ok, offA + sk + t, SENT + pos + 2 + t)
            plsc.store_scatter(idxg, [lane * NG + pos + 2 + t], it)
        sk = (sk >> 3) + 1
        ek = (ek >> 3) - 1
        pos += 8

    nonempty = sk <= ek
    for t in range(4):
        ok = jnp.logical_and(nonempty, sk + t <= ek)
        it = jnp.where(ok, OFF_A4 + sk + t, SENT + pos + t)
        plsc.store_scatter(idxg, [lane * NG + pos + t], it)
    for t in range(4):
        plsc.store_scatter(
            idxg, [lane * NG + pos + 4 + t],
            jnp.full((L,), SENT + pos + 4 + t, jnp.int32))

    semsA = (semA0, semA1)
    semsB = (semB0, semB1)

    def copies(j, p):
        cf = pltpu.make_async_copy(
            f_hbm.at[idx0.at[pl.ds(j * N0, N0)]],
            rows.at[p, pl.ds(0, N0)], semsA[p])
        cg = pltpu.make_async_copy(
            g_hbm.at[idxg.at[pl.ds(j * NG, NG)]],
            rows.at[p, pl.ds(N0, NG)], semsB[p])
        return cf, cg

    def start(j, p):
        cf, cg = copies(j, p)
        cf.start()
        cg.start()

    def wait(j, p):
        cf, cg = copies(j, p)
        cf.wait()
        cg.wait()

    def reduce_seg(j, p):
        def v_body(v, c):
            vals = [rows[p, r, pl.ds(v * L, L)] for r in range(NROWS)]
            while len(vals) > 1:
                nxt = [jnp.maximum(vals[i], vals[i + 1])
                       for i in range(0, len(vals) - 1, 2)]
                if len(vals) % 2:
                    nxt.append(vals[-1])
                vals = nxt
            outs[j, pl.ds(v * L, L)] = vals[0]
            return c
        lax.fori_loop(0, D // L, v_body, 0)

    start(0, 0)

    def body(i, c):
        j0 = 2 * i
        start(j0 + 1, 1)
        wait(j0, 0)
        reduce_seg(j0, 0)

        @pl.when(i < SEG_PER_W // 2 - 1)
        def _():
            start(j0 + 2, 0)

        wait(j0 + 1, 1)
        reduce_seg(j0 + 1, 1)
        return c

    lax.fori_loop(0, SEG_PER_W // 2, body, 0)
    pltpu.sync_copy(outs, out_hbm.at[pl.ds(base, SEG_PER_W)])


def _segmax(frame_features, g, s_arr, e_arr):
    mesh = plsc.VectorSubcoreMesh(core_axis_name="c", subcore_axis_name="s")
    run = functools.partial(
        pl.kernel,
        out_type=jax.ShapeDtypeStruct((S, D), jnp.float32),
        mesh=mesh,
        compiler_params=pltpu.CompilerParams(needs_layout_passes=False),
        scratch_types=[
            pltpu.VMEM((SEG_PER_W,), jnp.int32),
            pltpu.VMEM((SEG_PER_W,), jnp.int32),
            pltpu.VMEM((SEG_PER_W * N0,), jnp.int32),
            pltpu.VMEM((SEG_PER_W * NG,), jnp.int32),
            pltpu.VMEM((2, NROWS, D), jnp.float32),
            pltpu.VMEM((SEG_PER_W, D), jnp.float32),
            pltpu.SemaphoreType.DMA,
            pltpu.SemaphoreType.DMA,
            pltpu.SemaphoreType.DMA,
            pltpu.SemaphoreType.DMA,
        ],
    )(_segmax_body)
    return run(frame_features, g, s_arr, e_arr)


def _ln(x, gamma, beta):
    mu = jnp.mean(x, axis=-1, keepdims=True)
    var = jnp.mean((x - mu) ** 2, axis=-1, keepdims=True)
    return (x - mu) / jnp.sqrt(var + 1e-6) * gamma + beta


def _mlp_body(x_ref, w1_ref, b1_ref, w2_ref, b2_ref,
              gy_ref, by_ref, gl_ref, bl_ref, out_ref):
    x = x_ref[...]
    y = _ln(x, gy_ref[...], by_ref[...])
    h = jnp.dot(y, w1_ref[...], preferred_element_type=jnp.float32)
    h = jax.nn.relu(h + b1_ref[...])
    h = _ln(h, gl_ref[...], bl_ref[...])
    logits = jnp.sum(h * w2_ref[...], axis=-1) + b2_ref[0, 0]
    out_ref[...] = jax.nn.sigmoid(logits)[None, :]


def _mlp(segs, W1, b1, W2, b2, gamma_y, beta_y, gamma_l, beta_l):
    full = lambda i: (0, 0)
    return pl.pallas_call(
        _mlp_body,
        grid=(S // 128,),
        in_specs=[
            pl.BlockSpec((128, D), lambda i: (i, 0)),
            pl.BlockSpec((D, D), full),
            pl.BlockSpec((1, D), full),
            pl.BlockSpec((1, D), full),
            pl.BlockSpec((1, 1), full),
            pl.BlockSpec((1, D), full),
            pl.BlockSpec((1, D), full),
            pl.BlockSpec((1, D), full),
            pl.BlockSpec((1, D), full),
        ],
        out_specs=pl.BlockSpec((1, 128), lambda i: (0, i)),
        out_shape=jax.ShapeDtypeStruct((1, S), jnp.float32),
    )(segs, W1, b1.reshape(1, D), W2.reshape(1, D), b2.reshape(1, 1),
      gamma_y.reshape(1, D), beta_y.reshape(1, D),
      gamma_l.reshape(1, D), beta_l.reshape(1, D))


def kernel(frame_features, W1, b1, W2, b2,
           gamma_y, beta_y, gamma_l, beta_l, change_point):
    cp = change_point.astype(jnp.int32)
    s_arr = cp[:, 0]
    e_arr = cp[:, 1]
    g = _build_table(frame_features)
    segs = _segmax(frame_features, g, s_arr, e_arr)
    return _mlp(segs, W1, b1, W2, b2, gamma_y, beta_y, gamma_l, beta_l)


# speedup vs baseline: 2.0186x; 1.2013x over previous
import functools

import jax
import jax.numpy as jnp
from jax import lax
from jax.experimental import pallas as pl
from jax.experimental.pallas import tpu as pltpu
from jax.experimental.pallas import tpu_sc as plsc

T = 16384
D = 1024
S = 512

A1N, A2N, A3N, A4N = 2048, 256, 32, 4
OFF_A1 = 8
OFF_Q1 = OFF_A1 + A1N
OFF_P1 = OFF_Q1 + A1N
OFF_A2 = OFF_P1 + A1N
OFF_Q2 = OFF_A2 + A2N
OFF_P2 = OFF_Q2 + A2N
OFF_A3 = OFF_P2 + A2N
OFF_Q3 = OFF_A3 + A3N
OFF_P3 = OFF_Q3 + A3N
OFF_A4 = OFF_P3 + A3N
SENT = OFF_A4 + 8
G_ROWS = SENT + 32

NC, NS, L = 2, 16, 16
NW = NC * NS
SEG_PER_W = S // NW
N0 = 16
NG = 24
NROWS = N0 + NG


def _psa(a, n):
    x3 = a.reshape(n, 8, 128)
    pres = [x3[:, 0]]
    for i in range(1, 8):
        pres.append(jnp.maximum(pres[-1], x3[:, i]))
    sufs = [x3[:, 7]]
    for i in range(6, -1, -1):
        sufs.insert(0, jnp.maximum(sufs[0], x3[:, i]))
    p = jnp.stack(pres, axis=1).reshape(n * 8, 128)
    q = jnp.stack(sufs, axis=1).reshape(n * 8, 128)
    return p, q, pres[-1]


def _build_body(f_ref, g_ref):
    x = f_ref[...]
    a1 = jnp.max(x.reshape(A1N, 8, 128), axis=1)
    p1, q1, a2 = _psa(a1, A2N)
    p2, q2, a3 = _psa(a2, A3N)
    p3, q3, a4 = _psa(a3, A4N)
    neg8 = jnp.full((8, 128), -jnp.inf, jnp.float32)
    g_ref[0:OFF_A1, :] = neg8
    g_ref[OFF_A1:OFF_Q1, :] = a1
    g_ref[OFF_Q1:OFF_P1, :] = q1
    g_ref[OFF_P1:OFF_A2, :] = p1
    g_ref[OFF_A2:OFF_Q2, :] = a2
    g_ref[OFF_Q2:OFF_P2, :] = q2
    g_ref[OFF_P2:OFF_A3, :] = p2
    g_ref[OFF_A3:OFF_Q3, :] = a3
    g_ref[OFF_Q3:OFF_P3, :] = q3
    g_ref[OFF_P3:OFF_A4, :] = p3
    g_ref[OFF_A4:G_ROWS, :] = jnp.concatenate(
        [a4, jnp.full((G_ROWS - OFF_A4 - A4N, 128), -jnp.inf, jnp.float32)],
        axis=0)


def _build_table(frame_features):
    return pl.pallas_call(
        _build_body,
        grid=(D // 128,),
        in_specs=[pl.BlockSpec((T, 128), lambda i: (0, i))],
        out_specs=pl.BlockSpec((G_ROWS, 128), lambda i: (0, i)),
        out_shape=jax.ShapeDtypeStruct((G_ROWS, D), jnp.float32),
    )(frame_features)


_LEVELS = ((OFF_A1, OFF_Q1, OFF_P1),
           (OFF_A2, OFF_Q2, OFF_P2))


def _segmax_body(f_hbm, g_hbm, s_hbm, e_hbm, out_hbm,
                 sv, ev, idx0, idxg, rows, outs,
                 semA0, semB0, semA1, semB1):
    wid = lax.axis_index("s") * NC + lax.axis_index("c")
    base = wid * SEG_PER_W

    pltpu.sync_copy(s_hbm.at[pl.ds(base, SEG_PER_W)], sv)
    pltpu.sync_copy(e_hbm.at[pl.ds(base, SEG_PER_W)], ev)
    s0 = sv[...]
    e0 = ev[...]
    lane = lax.iota(jnp.int32, L)
    zero = jnp.zeros((L,), jnp.int32)

    for t in range(8):
        plsc.store_scatter(idx0, [lane * N0 + t], jnp.minimum(s0 + t, e0))
        plsc.store_scatter(idx0, [lane * N0 + (8 + t)], jnp.maximum(e0 - t, s0))

    sk = (s0 + 7) >> 3
    ek = ((e0 + 1) >> 3) - 1
    pos = 0
    for offA, offQ, offP in _LEVELS:
        nonempty = sk <= ek
        be_s = ((sk >> 3) << 3) + 7
        bs_e = (ek >> 3) << 3
        qv = jnp.logical_and(nonempty, be_s <= ek)
        pv = jnp.logical_and(nonempty, bs_e >= sk)
        interior = jnp.logical_and(
            nonempty,
            jnp.logical_and(jnp.logical_not(qv), jnp.logical_not(pv)))
        l0 = jnp.where(qv, offQ + sk, SENT + pos)
        l1 = jnp.where(pv, offP + ek, SENT + pos + 1)
        plsc.store_scatter(idxg, [lane * NG + pos], l0)
        plsc.store_scatter(idxg, [lane * NG + pos + 1], l1)
        for t in range(6):
            ok = jnp.logical_and(interior, sk + t <= ek)
            it = jnp.where(ok, offA + sk + t, SENT + pos + 2 + t)
            plsc.store_scatter(idxg, [lane * NG + pos + 2 + t], it)
        sk = (sk >> 3) + 1
        ek = (ek >> 3) - 1
        pos += 8

    s3, e3 = sk, ek
    nonempty3 = s3 <= e3
    be_s = ((s3 >> 3) << 3) + 7
    bs_e = (e3 >> 3) << 3
    qv3 = jnp.logical_and(nonempty3, be_s <= e3)
    pv3 = jnp.logical_and(nonempty3, bs_e >= s3)
    interior3 = jnp.logical_and(
        nonempty3,
        jnp.logical_and(jnp.logical_not(qv3), jnp.logical_not(pv3)))
    l0 = jnp.where(qv3, OFF_Q3 + s3, SENT + pos)
    l1 = jnp.where(pv3, OFF_P3 + e3, SENT + pos + 1)
    plsc.store_scatter(idxg, [lane * NG + pos], l0)
    plsc.store_scatter(idxg, [lane * NG + pos + 1], l1)
    s4 = (s3 >> 3) + 1
    e4 = (e3 >> 3) - 1
    nonempty4 = s4 <= e4
    for t in range(6):
        ok3 = jnp.logical_and(interior3, s3 + t <= e3)
        it = jnp.full((L,), SENT + pos + 2 + t, jnp.int32)
        if t < 4:
            ok4 = jnp.logical_and(nonempty4, s4 + t <= e4)
            it = jnp.where(ok4, OFF_A4 + s4 + t, it)
        it = jnp.where(ok3, OFF_A3 + s3 + t, it)
        plsc.store_scatter(idxg, [lane * NG + pos + 2 + t], it)

    semsA = (semA0, semA1)
    semsB = (semB0, semB1)

    def copies(j, p):
        cf = pltpu.make_async_copy(
            f_hbm.at[idx0.at[pl.ds(j * N0, N0)]],
            rows.at[p, pl.ds(0, N0)], semsA[p])
        cg = pltpu.make_async_copy(
            g_hbm.at[idxg.at[pl.ds(j * NG, NG)]],
            rows.at[p, pl.ds(N0, NG)], semsB[p])
        return cf, cg

    def start(j, p):
        cf, cg = copies(j, p)
        cf.start()
        cg.start()

    def wait(j, p):
        cf, cg = copies(j, p)
        cf.wait()
        cg.wait()

    def reduce_seg(j, p):
        def v_body(v, c):
            vals = [rows[p, r, pl.ds(v * L, L)] for r in range(NROWS)]
            while len(vals) > 1:
                nxt = [jnp.maximum(vals[i], vals[i + 1])
                       for i in range(0, len(vals) - 1, 2)]
                if len(vals) % 2:
                    nxt.append(vals[-1])
                vals = nxt
            outs[j, pl.ds(v * L, L)] = vals[0]
            return c
        lax.fori_loop(0, D // L, v_body, 0)

    start(0, 0)

    def body(i, c):
        j0 = 2 * i
        start(j0 + 1, 1)
        wait(j0, 0)
        reduce_seg(j0, 0)

        @pl.when(i < SEG_PER_W // 2 - 1)
        def _():
            start(j0 + 2, 0)

        wait(j0 + 1, 1)
        reduce_seg(j0 + 1, 1)
        return c

    lax.fori_loop(0, SEG_PER_W // 2, body, 0)
    pltpu.sync_copy(outs, out_hbm.at[pl.ds(base, SEG_PER_W)])


def _segmax(frame_features, g, s_arr, e_arr):
    mesh = plsc.VectorSubcoreMesh(core_axis_name="c", subcore_axis_name="s")
    run = functools.partial(
        pl.kernel,
        out_type=jax.ShapeDtypeStruct((S, D), jnp.float32),
        mesh=mesh,
        compiler_params=pltpu.CompilerParams(needs_layout_passes=False),
        scratch_types=[
            pltpu.VMEM((SEG_PER_W,), jnp.int32),
            pltpu.VMEM((SEG_PER_W,), jnp.int32),
            pltpu.VMEM((SEG_PER_W * N0,), jnp.int32),
            pltpu.VMEM((SEG_PER_W * NG,), jnp.int32),
            pltpu.VMEM((2, NROWS, D), jnp.float32),
            pltpu.VMEM((SEG_PER_W, D), jnp.float32),
            pltpu.SemaphoreType.DMA,
            pltpu.SemaphoreType.DMA,
            pltpu.SemaphoreType.DMA,
            pltpu.SemaphoreType.DMA,
        ],
    )(_segmax_body)
    return run(frame_features, g, s_arr, e_arr)


def _ln(x, gamma, beta):
    mu = jnp.mean(x, axis=-1, keepdims=True)
    var = jnp.mean((x - mu) ** 2, axis=-1, keepdims=True)
    return (x - mu) / jnp.sqrt(var + 1e-6) * gamma + beta


def _mlp_body(x_ref, w1_ref, b1_ref, w2_ref, b2_ref,
              gy_ref, by_ref, gl_ref, bl_ref, out_ref):
    x = x_ref[...]
    y = _ln(x, gy_ref[...], by_ref[...])
    h = jnp.dot(y, w1_ref[...], preferred_element_type=jnp.float32)
    h = jax.nn.relu(h + b1_ref[...])
    h = _ln(h, gl_ref[...], bl_ref[...])
    logits = jnp.sum(h * w2_ref[...], axis=-1) + b2_ref[0, 0]
    out_ref[...] = jax.nn.sigmoid(logits)[None, :]


def _mlp(segs, W1, b1, W2, b2, gamma_y, beta_y, gamma_l, beta_l):
    full = lambda i: (0, 0)
    return pl.pallas_call(
        _mlp_body,
        grid=(S // 128,),
        in_specs=[
            pl.BlockSpec((128, D), lambda i: (i, 0)),
            pl.BlockSpec((D, D), full),
            pl.BlockSpec((1, D), full),
            pl.BlockSpec((1, D), full),
            pl.BlockSpec((1, 1), full),
            pl.BlockSpec((1, D), full),
            pl.BlockSpec((1, D), full),
            pl.BlockSpec((1, D), full),
            pl.BlockSpec((1, D), full),
        ],
        out_specs=pl.BlockSpec((1, 128), lambda i: (0, i)),
        out_shape=jax.ShapeDtypeStruct((1, S), jnp.float32),
    )(segs, W1, b1.reshape(1, D), W2.reshape(1, D), b2.reshape(1, 1),
      gamma_y.reshape(1, D), beta_y.reshape(1, D),
      gamma_l.reshape(1, D), beta_l.reshape(1, D))


def kernel(frame_features, W1, b1, W2, b2,
           gamma_y, beta_y, gamma_l, beta_l, change_point):
    cp = change_point.astype(jnp.int32)
    s_arr = cp[:, 0]
    e_arr = cp[:, 1]
    g = _build_table(frame_features)
    segs = _segmax(frame_features, g, s_arr, e_arr)
    return _mlp(segs, W1, b1, W2, b2, gamma_y, beta_y, gamma_l, beta_l)


# speedup vs baseline: 2.1295x; 1.0550x over previous
import functools

import jax
import jax.numpy as jnp
from jax import lax
from jax.experimental import pallas as pl
from jax.experimental.pallas import tpu as pltpu
from jax.experimental.pallas import tpu_sc as plsc

T = 16384
D = 1024
S = 512

A1N, A2N, A3N, A4N = 2048, 256, 32, 4
OFF_A1 = 8
OFF_Q1 = OFF_A1 + A1N
OFF_P1 = OFF_Q1 + A1N
OFF_A2 = OFF_P1 + A1N
OFF_Q2 = OFF_A2 + A2N
OFF_P2 = OFF_Q2 + A2N
OFF_A3 = OFF_P2 + A2N
OFF_Q3 = OFF_A3 + A3N
OFF_P3 = OFF_Q3 + A3N
OFF_A4 = OFF_P3 + A3N
SENT = OFF_A4 + 8
G_ROWS = SENT + 32

NC, NS, L = 2, 16, 16
NW = NC * NS
SEG_PER_W = S // NW
N0 = 16
NG = 24
NROWS = N0 + NG


def _psa(a, n):
    x3 = a.reshape(n, 8, 128)
    pres = [x3[:, 0]]
    for i in range(1, 8):
        pres.append(jnp.maximum(pres[-1], x3[:, i]))
    sufs = [x3[:, 7]]
    for i in range(6, -1, -1):
        sufs.insert(0, jnp.maximum(sufs[0], x3[:, i]))
    p = jnp.stack(pres, axis=1).reshape(n * 8, 128)
    q = jnp.stack(sufs, axis=1).reshape(n * 8, 128)
    return p, q, pres[-1]


def _build_body(f_ref, g_ref):
    x = f_ref[...]
    a1 = jnp.max(x.reshape(A1N, 8, 128), axis=1)
    p1, q1, a2 = _psa(a1, A2N)
    p2, q2, a3 = _psa(a2, A3N)
    p3, q3, a4 = _psa(a3, A4N)
    neg8 = jnp.full((8, 128), -jnp.inf, jnp.float32)
    g_ref[0:OFF_A1, :] = neg8
    g_ref[OFF_A1:OFF_Q1, :] = a1
    g_ref[OFF_Q1:OFF_P1, :] = q1
    g_ref[OFF_P1:OFF_A2, :] = p1
    g_ref[OFF_A2:OFF_Q2, :] = a2
    g_ref[OFF_Q2:OFF_P2, :] = q2
    g_ref[OFF_P2:OFF_A3, :] = p2
    g_ref[OFF_A3:OFF_Q3, :] = a3
    g_ref[OFF_Q3:OFF_P3, :] = q3
    g_ref[OFF_P3:OFF_A4, :] = p3
    g_ref[OFF_A4:G_ROWS, :] = jnp.concatenate(
        [a4, jnp.full((G_ROWS - OFF_A4 - A4N, 128), -jnp.inf, jnp.float32)],
        axis=0)


def _build_table(frame_features):
    return pl.pallas_call(
        _build_body,
        grid=(D // 128,),
        in_specs=[pl.BlockSpec((T, 128), lambda i: (0, i))],
        out_specs=pl.BlockSpec((G_ROWS, 128), lambda i: (0, i)),
        out_shape=jax.ShapeDtypeStruct((G_ROWS, D), jnp.float32),
    )(frame_features)


_LEVELS = ((OFF_A1, OFF_Q1, OFF_P1),
           (OFF_A2, OFF_Q2, OFF_P2))


def _segmax_body(f_hbm, g_hbm, s_hbm, e_hbm, out_hbm,
                 sv, ev, idx0, idxg, rows, outs,
                 semA0, semB0, semA1, semB1):
    wid = lax.axis_index("s") * NC + lax.axis_index("c")
    base = wid * SEG_PER_W

    pltpu.sync_copy(s_hbm.at[pl.ds(base, SEG_PER_W)], sv)
    pltpu.sync_copy(e_hbm.at[pl.ds(base, SEG_PER_W)], ev)
    s0 = sv[...]
    e0 = ev[...]
    lane = lax.iota(jnp.int32, L)

    def snt(pos_off):
        return SENT + ((pos_off + lane * 8) & 31)

    for t in range(8):
        plsc.store_scatter(idx0, [lane * N0 + t], jnp.minimum(s0 + t, e0))
        plsc.store_scatter(idx0, [lane * N0 + (15 - t)], jnp.maximum(e0 - t, s0))

    sk = (s0 + 7) >> 3
    ek = ((e0 + 1) >> 3) - 1
    pos = 0
    for offA, offQ, offP in _LEVELS:
        nonempty = sk <= ek
        be_s = ((sk >> 3) << 3) + 7
        bs_e = (ek >> 3) << 3
        qv = jnp.logical_and(nonempty, be_s <= ek)
        pv = jnp.logical_and(nonempty, bs_e >= sk)
        interior = jnp.logical_and(
            nonempty,
            jnp.logical_and(jnp.logical_not(qv), jnp.logical_not(pv)))
        l0 = jnp.where(qv, offQ + sk, snt(pos))
        l1 = jnp.where(pv, offP + ek, snt(pos + 1))
        plsc.store_scatter(idxg, [lane * NG + pos], l0)
        plsc.store_scatter(idxg, [lane * NG + pos + 1], l1)
        for t in range(6):
            ok = jnp.logical_and(interior, sk + t <= ek)
            it = jnp.where(ok, offA + sk + t, snt(pos + 2 + t))
            plsc.store_scatter(idxg, [lane * NG + pos + 2 + t], it)
        sk = (sk >> 3) + 1
        ek = (ek >> 3) - 1
        pos += 8

    s3, e3 = sk, ek
    nonempty3 = s3 <= e3
    be_s = ((s3 >> 3) << 3) + 7
    bs_e = (e3 >> 3) << 3
    qv3 = jnp.logical_and(nonempty3, be_s <= e3)
    pv3 = jnp.logical_and(nonempty3, bs_e >= s3)
    interior3 = jnp.logical_and(
        nonempty3,
        jnp.logical_and(jnp.logical_not(qv3), jnp.logical_not(pv3)))
    l0 = jnp.where(qv3, OFF_Q3 + s3, snt(pos))
    l1 = jnp.where(pv3, OFF_P3 + e3, snt(pos + 1))
    plsc.store_scatter(idxg, [lane * NG + pos], l0)
    plsc.store_scatter(idxg, [lane * NG + pos + 1], l1)
    s4 = (s3 >> 3) + 1
    e4 = (e3 >> 3) - 1
    nonempty4 = s4 <= e4
    for t in range(6):
        ok3 = jnp.logical_and(interior3, s3 + t <= e3)
        it = snt(pos + 2 + t)
        if t < 4:
            ok4 = jnp.logical_and(nonempty4, s4 + t <= e4)
            it = jnp.where(ok4, OFF_A4 + s4 + t, it)
        it = jnp.where(ok3, OFF_A3 + s3 + t, it)
        plsc.store_scatter(idxg, [lane * NG + pos + 2 + t], it)

    semsA = (semA0, semA1)
    semsB = (semB0, semB1)

    def copies(j, p):
        cf = pltpu.make_async_copy(
            f_hbm.at[idx0.at[pl.ds(j * N0, N0)]],
            rows.at[p, pl.ds(0, N0)], semsA[p])
        cg = pltpu.make_async_copy(
            g_hbm.at[idxg.at[pl.ds(j * NG, NG)]],
            rows.at[p, pl.ds(N0, NG)], semsB[p])
        return cf, cg

    def start(j, p):
        cf, cg = copies(j, p)
        cf.start()
        cg.start()

    def wait(j, p):
        cf, cg = copies(j, p)
        cf.wait()
        cg.wait()

    def reduce_seg(j, p):
        def v_body(v, c):
            vals = [rows[p, r, pl.ds(v * L, L)] for r in range(NROWS)]
            while len(vals) > 1:
                nxt = [jnp.maximum(vals[i], vals[i + 1])
                       for i in range(0, len(vals) - 1, 2)]
                if len(vals) % 2:
                    nxt.append(vals[-1])
                vals = nxt
            outs[j, pl.ds(v * L, L)] = vals[0]
            return c
        lax.fori_loop(0, D // L, v_body, 0)

    start(0, 0)

    def body(i, c):
        j0 = 2 * i
        start(j0 + 1, 1)
        wait(j0, 0)
        reduce_seg(j0, 0)

        @pl.when(i < SEG_PER_W // 2 - 1)
        def _():
            start(j0 + 2, 0)

        wait(j0 + 1, 1)
        reduce_seg(j0 + 1, 1)
        return c

    lax.fori_loop(0, SEG_PER_W // 2, body, 0)
    pltpu.sync_copy(outs, out_hbm.at[pl.ds(base, SEG_PER_W)])


def _segmax(frame_features, g, s_arr, e_arr):
    mesh = plsc.VectorSubcoreMesh(core_axis_name="c", subcore_axis_name="s")
    run = functools.partial(
        pl.kernel,
        out_type=jax.ShapeDtypeStruct((S, D), jnp.float32),
        mesh=mesh,
        compiler_params=pltpu.CompilerParams(needs_layout_passes=False),
        scratch_types=[
            pltpu.VMEM((SEG_PER_W,), jnp.int32),
            pltpu.VMEM((SEG_PER_W,), jnp.int32),
            pltpu.VMEM((SEG_PER_W * N0,), jnp.int32),
            pltpu.VMEM((SEG_PER_W * NG,), jnp.int32),
            pltpu.VMEM((2, NROWS, D), jnp.float32),
            pltpu.VMEM((SEG_PER_W, D), jnp.float32),
            pltpu.SemaphoreType.DMA,
            pltpu.SemaphoreType.DMA,
            pltpu.SemaphoreType.DMA,
            pltpu.SemaphoreType.DMA,
        ],
    )(_segmax_body)
    return run(frame_features, g, s_arr, e_arr)


def _ln(x, gamma, beta):
    mu = jnp.mean(x, axis=-1, keepdims=True)
    var = jnp.mean((x - mu) ** 2, axis=-1, keepdims=True)
    return (x - mu) / jnp.sqrt(var + 1e-6) * gamma + beta


def _mlp_body(x_ref, w1_ref, b1_ref, w2_ref, b2_ref,
              gy_ref, by_ref, gl_ref, bl_ref, out_ref):
    x = x_ref[...]
    y = _ln(x, gy_ref[...], by_ref[...])
    h = jnp.dot(y, w1_ref[...], preferred_element_type=jnp.float32)
    h = jax.nn.relu(h + b1_ref[...])
    h = _ln(h, gl_ref[...], bl_ref[...])
    logits = jnp.sum(h * w2_ref[...], axis=-1) + b2_ref[0, 0]
    out_ref[...] = jax.nn.sigmoid(logits)[None, :]


def _mlp(segs, W1, b1, W2, b2, gamma_y, beta_y, gamma_l, beta_l):
    full = lambda i: (0, 0)
    return pl.pallas_call(
        _mlp_body,
        grid=(S // 128,),
        in_specs=[
            pl.BlockSpec((128, D), lambda i: (i, 0)),
            pl.BlockSpec((D, D), full),
            pl.BlockSpec((1, D), full),
            pl.BlockSpec((1, D), full),
            pl.BlockSpec((1, 1), full),
            pl.BlockSpec((1, D), full),
            pl.BlockSpec((1, D), full),
            pl.BlockSpec((1, D), full),
            pl.BlockSpec((1, D), full),
        ],
        out_specs=pl.BlockSpec((1, 128), lambda i: (0, i)),
        out_shape=jax.ShapeDtypeStruct((1, S), jnp.float32),
    )(segs, W1, b1.reshape(1, D), W2.reshape(1, D), b2.reshape(1, 1),
      gamma_y.reshape(1, D), beta_y.reshape(1, D),
      gamma_l.reshape(1, D), beta_l.reshape(1, D))


def kernel(frame_features, W1, b1, W2, b2,
           gamma_y, beta_y, gamma_l, beta_l, change_point):
    cp = change_point.astype(jnp.int32)
    s_arr = cp[:, 0]
    e_arr = cp[:, 1]
    g = _build_table(frame_features)
    segs = _segmax(frame_features, g, s_arr, e_arr)
    return _mlp(segs, W1, b1, W2, b2, gamma_y, beta_y, gamma_l, beta_l)


# speedup vs baseline: 2.1374x; 1.0037x over previous
import functools

import jax
import jax.numpy as jnp
from jax import lax
from jax.experimental import pallas as pl
from jax.experimental.pallas import tpu as pltpu
from jax.experimental.pallas import tpu_sc as plsc

T = 16384
D = 1024
S = 512

A1N, A2N, A3N, A4N = 2048, 256, 32, 4
OFF_A1 = 8
OFF_Q1 = OFF_A1 + A1N
OFF_P1 = OFF_Q1 + A1N
OFF_A2 = OFF_P1 + A1N
OFF_Q2 = OFF_A2 + A2N
OFF_P2 = OFF_Q2 + A2N
OFF_A3 = OFF_P2 + A2N
OFF_Q3 = OFF_A3 + A3N
OFF_P3 = OFF_Q3 + A3N
OFF_A4 = OFF_P3 + A3N
SENT = OFF_A4 + 8
G_ROWS = SENT + 32

NC, NS, L = 2, 16, 16
NW = NC * NS
SEG_PER_W = S // NW
N0 = 16
NG = 24
NROWS = N0 + NG


def _psa(a, n):
    x3 = a.reshape(n, 8, 128)
    pres = [x3[:, 0]]
    for i in range(1, 8):
        pres.append(jnp.maximum(pres[-1], x3[:, i]))
    sufs = [x3[:, 7]]
    for i in range(6, -1, -1):
        sufs.insert(0, jnp.maximum(sufs[0], x3[:, i]))
    p = jnp.stack(pres, axis=1).reshape(n * 8, 128)
    q = jnp.stack(sufs, axis=1).reshape(n * 8, 128)
    return p, q, pres[-1]


def _build_body(f_ref, g_ref):
    x = f_ref[...]
    a1 = jnp.max(x.reshape(A1N, 8, 128), axis=1)
    p1, q1, a2 = _psa(a1, A2N)
    p2, q2, a3 = _psa(a2, A3N)
    p3, q3, a4 = _psa(a3, A4N)
    neg8 = jnp.full((8, 128), -jnp.inf, jnp.float32)
    g_ref[0:OFF_A1, :] = neg8
    g_ref[OFF_A1:OFF_Q1, :] = a1
    g_ref[OFF_Q1:OFF_P1, :] = q1
    g_ref[OFF_P1:OFF_A2, :] = p1
    g_ref[OFF_A2:OFF_Q2, :] = a2
    g_ref[OFF_Q2:OFF_P2, :] = q2
    g_ref[OFF_P2:OFF_A3, :] = p2
    g_ref[OFF_A3:OFF_Q3, :] = a3
    g_ref[OFF_Q3:OFF_P3, :] = q3
    g_ref[OFF_P3:OFF_A4, :] = p3
    g_ref[OFF_A4:G_ROWS, :] = jnp.concatenate(
        [a4, jnp.full((G_ROWS - OFF_A4 - A4N, 128), -jnp.inf, jnp.float32)],
        axis=0)


def _build_table(frame_features):
    return pl.pallas_call(
        _build_body,
        grid=(D // 128,),
        in_specs=[pl.BlockSpec((T, 128), lambda i: (0, i))],
        out_specs=pl.BlockSpec((G_ROWS, 128), lambda i: (0, i)),
        out_shape=jax.ShapeDtypeStruct((G_ROWS, D), jnp.float32),
    )(frame_features)


_LEVELS = ((OFF_A1, OFF_Q1, OFF_P1),
           (OFF_A2, OFF_Q2, OFF_P2))


def _segmax_body(f_hbm, g_hbm, s_hbm, e_hbm, out_hbm,
                 sv, ev, idx0, idxg, rows, outs,
                 semA0, semB0, semA1, semB1):
    wid = lax.axis_index("s") * NC + lax.axis_index("c")
    base = wid * SEG_PER_W

    pltpu.sync_copy(s_hbm.at[pl.ds(base, SEG_PER_W)], sv)
    pltpu.sync_copy(e_hbm.at[pl.ds(base, SEG_PER_W)], ev)
    s0 = sv[...]
    e0 = ev[...]
    lane = lax.iota(jnp.int32, L)

    def snt(pos_off):
        return SENT + ((pos_off + lane * 8) & 31)

    for t in range(8):
        plsc.store_scatter(idx0, [lane * N0 + t], jnp.minimum(s0 + t, e0))
        plsc.store_scatter(idx0, [lane * N0 + (15 - t)], jnp.maximum(e0 - t, s0))

    sk = (s0 + 7) >> 3
    ek = ((e0 + 1) >> 3) - 1
    pos = 0
    for offA, offQ, offP in _LEVELS:
        nonempty = sk <= ek
        be_s = ((sk >> 3) << 3) + 7
        bs_e = (ek >> 3) << 3
        qv = jnp.logical_and(nonempty, be_s <= ek)
        pv = jnp.logical_and(nonempty, bs_e >= sk)
        interior = jnp.logical_and(
            nonempty,
            jnp.logical_and(jnp.logical_not(qv), jnp.logical_not(pv)))
        l0 = jnp.where(qv, offQ + sk, snt(pos))
        l1 = jnp.where(pv, offP + ek, snt(pos + 1))
        plsc.store_scatter(idxg, [lane * NG + pos], l0)
        plsc.store_scatter(idxg, [lane * NG + pos + 1], l1)
        for t in range(6):
            ok = jnp.logical_and(interior, sk + t <= ek)
            it = jnp.where(ok, offA + sk + t, snt(pos + 2 + t))
            plsc.store_scatter(idxg, [lane * NG + pos + 2 + t], it)
        sk = (sk >> 3) + 1
        ek = (ek >> 3) - 1
        pos += 8

    s3, e3 = sk, ek
    nonempty3 = s3 <= e3
    be_s = ((s3 >> 3) << 3) + 7
    bs_e = (e3 >> 3) << 3
    qv3 = jnp.logical_and(nonempty3, be_s <= e3)
    pv3 = jnp.logical_and(nonempty3, bs_e >= s3)
    interior3 = jnp.logical_and(
        nonempty3,
        jnp.logical_and(jnp.logical_not(qv3), jnp.logical_not(pv3)))
    l0 = jnp.where(qv3, OFF_Q3 + s3, snt(pos))
    l1 = jnp.where(pv3, OFF_P3 + e3, snt(pos + 1))
    plsc.store_scatter(idxg, [lane * NG + pos], l0)
    plsc.store_scatter(idxg, [lane * NG + pos + 1], l1)
    s4 = (s3 >> 3) + 1
    e4 = (e3 >> 3) - 1
    nonempty4 = s4 <= e4
    for t in range(6):
        ok3 = jnp.logical_and(interior3, s3 + t <= e3)
        it = snt(pos + 2 + t)
        if t < 4:
            ok4 = jnp.logical_and(nonempty4, s4 + t <= e4)
            it = jnp.where(ok4, OFF_A4 + s4 + t, it)
        it = jnp.where(ok3, OFF_A3 + s3 + t, it)
        plsc.store_scatter(idxg, [lane * NG + pos + 2 + t], it)

    semsA = (semA0, semA1)
    semsB = (semB0, semB1)

    def copies(j, p):
        cf = pltpu.make_async_copy(
            f_hbm.at[idx0.at[pl.ds(j * N0, N0)]],
            rows.at[p, pl.ds(0, N0)], semsA[p])
        cg = pltpu.make_async_copy(
            g_hbm.at[idxg.at[pl.ds(j * NG, NG)]],
            rows.at[p, pl.ds(N0, NG)], semsB[p])
        return cf, cg

    def start(j, p):
        cf, cg = copies(j, p)
        cf.start()
        cg.start()

    def _tree_max(vals):
        while len(vals) > 1:
            nxt = [jnp.maximum(vals[i], vals[i + 1])
                   for i in range(0, len(vals) - 1, 2)]
            if len(vals) % 2:
                nxt.append(vals[-1])
            vals = nxt
        return vals[0]

    def reduce_seg(j, p):
        cf, cg = copies(j, p)
        cf.wait()

        def vf_body(v, c):
            vals = [rows[p, r, pl.ds(v * L, L)] for r in range(N0)]
            outs[j, pl.ds(v * L, L)] = _tree_max(vals)
            return c
        lax.fori_loop(0, D // L, vf_body, 0)

        cg.wait()

        def vg_body(v, c):
            vals = [rows[p, r, pl.ds(v * L, L)] for r in range(N0, NROWS)]
            vals.append(outs[j, pl.ds(v * L, L)])
            outs[j, pl.ds(v * L, L)] = _tree_max(vals)
            return c
        lax.fori_loop(0, D // L, vg_body, 0)

    start(0, 0)

    def body(i, c):
        j0 = 2 * i
        start(j0 + 1, 1)
        reduce_seg(j0, 0)

        @pl.when(i < SEG_PER_W // 2 - 1)
        def _():
            start(j0 + 2, 0)

        reduce_seg(j0 + 1, 1)
        return c

    lax.fori_loop(0, SEG_PER_W // 2, body, 0)
    pltpu.sync_copy(outs, out_hbm.at[pl.ds(base, SEG_PER_W)])


def _segmax(frame_features, g, s_arr, e_arr):
    mesh = plsc.VectorSubcoreMesh(core_axis_name="c", subcore_axis_name="s")
    run = functools.partial(
        pl.kernel,
        out_type=jax.ShapeDtypeStruct((S, D), jnp.float32),
        mesh=mesh,
        compiler_params=pltpu.CompilerParams(needs_layout_passes=False),
        scratch_types=[
            pltpu.VMEM((SEG_PER_W,), jnp.int32),
            pltpu.VMEM((SEG_PER_W,), jnp.int32),
            pltpu.VMEM((SEG_PER_W * N0,), jnp.int32),
            pltpu.VMEM((SEG_PER_W * NG,), jnp.int32),
            pltpu.VMEM((2, NROWS, D), jnp.float32),
            pltpu.VMEM((SEG_PER_W, D), jnp.float32),
            pltpu.SemaphoreType.DMA,
            pltpu.SemaphoreType.DMA,
            pltpu.SemaphoreType.DMA,
            pltpu.SemaphoreType.DMA,
        ],
    )(_segmax_body)
    return run(frame_features, g, s_arr, e_arr)


def _ln(x, gamma, beta):
    mu = jnp.mean(x, axis=-1, keepdims=True)
    var = jnp.mean((x - mu) ** 2, axis=-1, keepdims=True)
    return (x - mu) / jnp.sqrt(var + 1e-6) * gamma + beta


def _mlp_body(x_ref, w1_ref, b1_ref, w2_ref, b2_ref,
              gy_ref, by_ref, gl_ref, bl_ref, out_ref):
    x = x_ref[...]
    y = _ln(x, gy_ref[...], by_ref[...])
    h = jnp.dot(y, w1_ref[...], preferred_element_type=jnp.float32)
    h = jax.nn.relu(h + b1_ref[...])
    h = _ln(h, gl_ref[...], bl_ref[...])
    logits = jnp.sum(h * w2_ref[...], axis=-1) + b2_ref[0, 0]
    out_ref[...] = jax.nn.sigmoid(logits)[None, :]


def _mlp(segs, W1, b1, W2, b2, gamma_y, beta_y, gamma_l, beta_l):
    full = lambda i: (0, 0)
    return pl.pallas_call(
        _mlp_body,
        grid=(S // 128,),
        in_specs=[
            pl.BlockSpec((128, D), lambda i: (i, 0)),
            pl.BlockSpec((D, D), full),
            pl.BlockSpec((1, D), full),
            pl.BlockSpec((1, D), full),
            pl.BlockSpec((1, 1), full),
            pl.BlockSpec((1, D), full),
            pl.BlockSpec((1, D), full),
            pl.BlockSpec((1, D), full),
            pl.BlockSpec((1, D), full),
        ],
        out_specs=pl.BlockSpec((1, 128), lambda i: (0, i)),
        out_shape=jax.ShapeDtypeStruct((1, S), jnp.float32),
    )(segs, W1, b1.reshape(1, D), W2.reshape(1, D), b2.reshape(1, 1),
      gamma_y.reshape(1, D), beta_y.reshape(1, D),
      gamma_l.reshape(1, D), beta_l.reshape(1, D))


def kernel(frame_features, W1, b1, W2, b2,
           gamma_y, beta_y, gamma_l, beta_l, change_point):
    cp = change_point.astype(jnp.int32)
    s_arr = cp[:, 0]
    e_arr = cp[:, 1]
    g = _build_table(frame_features)
    segs = _segmax(frame_features, g, s_arr, e_arr)
    return _mlp(segs, W1, b1, W2, b2, gamma_y, beta_y, gamma_l, beta_l)
